# Initial kernel scaffold; baseline (speedup 1.0000x reference)
import jax
import jax.numpy as jnp
from jax.experimental import pallas as pl


def kernel(x, params):
    raise NotImplementedError("write your pallas kernel here")


# speedup vs baseline: 17.8115x; 17.8115x over previous
import jax
import jax.numpy as jnp
from jax.experimental import pallas as pl
from jax.experimental.pallas import tpu as pltpu

_DIMS = [10, 9, 8, 7, 6, 5, 4, 3, 2, 1]
_NLAYERS = len(_DIMS) - 1
_BIAS_COL = _DIMS[0]
_PSUB = 16
_PLANE = 128
_BBLK = 8192


def _mlp_kernel(x_ref, p_ref, o_ref):
    x = x_ref[...]
    wb0 = p_ref[0]
    w0 = wb0[:_DIMS[1], :_DIMS[0]]
    b0 = wb0[:_DIMS[1], _BIAS_COL:_BIAS_COL + 1]
    z = jax.lax.dot_general(
        w0, x, (((1,), (1,)), ((), ())),
        preferred_element_type=jnp.float32) + b0
    h = jnp.tanh(z)
    for layer in range(1, _NLAYERS):
        in_d, out_d = _DIMS[layer], _DIMS[layer + 1]
        wb = p_ref[layer]
        w = wb[:out_d, :in_d]
        b = wb[:out_d, _BIAS_COL:_BIAS_COL + 1]
        z = jnp.dot(w, h, preferred_element_type=jnp.float32) + b
        h = jax.nn.sigmoid(z)
    o_ref[...] = h


def _forward(x, params, bblk):
    batch = x.shape[0]
    out_t = pl.pallas_call(
        _mlp_kernel,
        out_shape=jax.ShapeDtypeStruct((1, batch), jnp.float32),
        grid=(batch // bblk,),
        in_specs=[
            pl.BlockSpec((bblk, _DIMS[0]), lambda i: (i, 0)),
            pl.BlockSpec((_NLAYERS, _PSUB, _PLANE), lambda i: (0, 0, 0)),
        ],
        out_specs=pl.BlockSpec((1, bblk), lambda i: (0, i)),
        compiler_params=pltpu.CompilerParams(
            dimension_semantics=("parallel",)),
    )(x, params)
    return jnp.transpose(out_t)


def kernel(x, params):
    batch = x.shape[0]
    bblk = None
    for cand in (_BBLK, 4096, 2048, 1024, 512, 256, 128):
        if batch % cand == 0:
            bblk = cand
            break
    if bblk is None:
        pad = (-batch) % 128
        xp = jnp.pad(x, ((0, pad), (0, 0)))
        return _forward(xp, params, 128)[:batch]
    return _forward(x, params, bblk)


# speedup vs baseline: 21.0433x; 1.1814x over previous
import jax
import jax.numpy as jnp
from jax.experimental import pallas as pl
from jax.experimental.pallas import tpu as pltpu

_DIMS = [10, 9, 8, 7, 6, 5, 4, 3, 2, 1]
_NLAYERS = len(_DIMS) - 1
_BIAS_COL = _DIMS[0]
_PSUB = 16
_PLANE = 128
_BBLK = 32768


def _mlp_kernel(x_ref, p_ref, o_ref):
    x = x_ref[...]
    wb0 = p_ref[0]
    w0 = wb0[:_DIMS[1], :_DIMS[0]]
    b0 = wb0[:_DIMS[1], _BIAS_COL:_BIAS_COL + 1]
    z = jax.lax.dot_general(
        w0, x, (((1,), (1,)), ((), ())),
        preferred_element_type=jnp.float32) + b0
    h = jnp.tanh(z)
    for layer in range(1, _NLAYERS):
        in_d, out_d = _DIMS[layer], _DIMS[layer + 1]
        wb = p_ref[layer]
        w = wb[:out_d, :in_d]
        b = wb[:out_d, _BIAS_COL:_BIAS_COL + 1]
        z = jnp.dot(w, h, preferred_element_type=jnp.float32) + b
        h = jax.nn.sigmoid(z)
    o_ref[...] = h[0]


def _forward(x, params, bblk):
    batch = x.shape[0]
    out_flat = pl.pallas_call(
        _mlp_kernel,
        out_shape=jax.ShapeDtypeStruct((batch,), jnp.float32),
        grid=(batch // bblk,),
        in_specs=[
            pl.BlockSpec((bblk, _DIMS[0]), lambda i: (i, 0)),
            pl.BlockSpec((_NLAYERS, _PSUB, _PLANE), lambda i: (0, 0, 0)),
        ],
        out_specs=pl.BlockSpec((bblk,), lambda i: (i,)),
        compiler_params=pltpu.CompilerParams(
            dimension_semantics=("parallel",)),
    )(x, params)
    return out_flat[:, None]


def kernel(x, params):
    batch = x.shape[0]
    bblk = None
    for cand in (_BBLK, 8192, 4096, 2048, 1024, 512, 256, 128):
        if batch % cand == 0:
            bblk = cand
            break
    if bblk is None:
        pad = (-batch) % 128
        xp = jnp.pad(x, ((0, pad), (0, 0)))
        return _forward(xp, params, 128)[:batch]
    return _forward(x, params, bblk)
